# Initial kernel scaffold; baseline (speedup 1.0000x reference)
import jax
import jax.numpy as jnp
from jax.experimental import pallas as pl


def kernel(x, W1, b1, g1, be1, W2, b2, g2, be2, W3, b3, g3, be3, Wc1, bc1, Wc2, bc2, edge_index):
    raise NotImplementedError("write your pallas kernel here")


# speedup vs baseline: 20.7724x; 20.7724x over previous
import functools

import jax
import jax.numpy as jnp
from jax import lax
from jax.experimental import pallas as pl
from jax.experimental.pallas import tpu as pltpu
from jax.experimental.pallas import tpu_sc as plsc

N_NODES = 10000
N_PAD = 10240
N_EDGES = 320000
IN_DIM = 128
HID = 256

NC = 2
NS = 16
CH = 128
RB = 40
E_PAD = 327680
ROWS_PER_TILE = N_PAD // NS
BLK = 2000
GRID = N_NODES // BLK


def _mesh():
    return plsc.VectorSubcoreMesh(core_axis_name="c", subcore_axis_name="s",
                                  num_cores=NC, num_subcores=NS)


def _pipeline_block(zref, acc, srcv, dstv, rows, gsem, ssem):
    def gather(j, k):
        pltpu.async_copy(zref.at[srcv.at[j]], rows[k], gsem[k])

    def gwait(j, k):
        pltpu.make_async_copy(zref.at[srcv.at[j]], rows[k], gsem[k]).wait()

    def scat(j, k):
        pltpu.async_copy(rows[k], acc.at[dstv.at[j]], ssem[k], add=True)

    def swait(j, k):
        pltpu.make_async_copy(rows[k], acc.at[dstv.at[j]], ssem[k]).wait()

    gather(0, 0)

    def pair(jj, _):
        j0 = jj * 2

        @pl.when(jj > 0)
        def _():
            swait(j0, 1)
        gather(j0 + 1, 1)
        gwait(j0, 0)
        scat(j0, 0)

        @pl.when(jj + 1 < RB // 2)
        def _():
            swait(j0, 0)
            gather(j0 + 2, 0)
        gwait(j0 + 1, 1)
        scat(j0 + 1, 1)
        return 0
    lax.fori_loop(0, RB // 2, pair, 0)
    swait(RB - 2, 0)
    swait(RB - 1, 1)


def _sc_degree(dst4, zeros):
    nb = E_PAD // (NC * NS * RB * CH)

    @functools.partial(
        pl.kernel,
        out_type=[jax.ShapeDtypeStruct((N_PAD, 128), jnp.float32)
                  for _ in range(NC)],
        mesh=_mesh(),
        scratch_types=[
            pltpu.VMEM_SHARED((N_PAD, 128), jnp.float32),
            pltpu.VMEM((RB, CH), jnp.int32),
            pltpu.VMEM((CH, 128), jnp.float32),
            pltpu.SemaphoreType.DMA,
        ],
    )
    def k(dst_hbm, zeros_hbm, out0, out1, acc, dstv, ones, ssem):
        c = lax.axis_index("c")
        s = lax.axis_index("s")
        w = c * NS + s

        def init_ones(i, _):
            for q in range(128 // 16):
                ones[i, pl.ds(q * 16, 16)] = jnp.ones((16,), jnp.float32)
            return 0
        lax.fori_loop(0, CH, init_ones, 0)

        my_rows = pl.ds(s * ROWS_PER_TILE, ROWS_PER_TILE)
        pltpu.sync_copy(zeros_hbm.at[my_rows], acc.at[my_rows])
        plsc.subcore_barrier()

        for b in range(nb):
            pltpu.sync_copy(dst_hbm.at[w, b], dstv)

            def fire(j, _):
                pltpu.async_copy(ones, acc.at[dstv.at[j]], ssem, add=True)
                return 0
            lax.fori_loop(0, RB, fire, 0)

            def drain(j, _):
                pltpu.make_async_copy(ones, acc.at[dstv.at[0]], ssem).wait()
                return 0
            lax.fori_loop(0, RB, drain, 0)

        plsc.subcore_barrier()
        for cc, out in enumerate((out0, out1)):
            @pl.when(c == cc)
            def _():
                pltpu.sync_copy(acc.at[my_rows], out.at[my_rows])

    return k(dst4, zeros)


def _sc_aggregate_es(z, zeros, src4, dst4):
    nb = E_PAD // (NC * NS * RB * CH)

    @functools.partial(
        pl.kernel,
        out_type=[jax.ShapeDtypeStruct((N_PAD, 128), jnp.float32)
                  for _ in range(NC)],
        mesh=_mesh(),
        scratch_types=[
            pltpu.VMEM_SHARED((N_PAD, 128), jnp.float32),
            pltpu.VMEM((RB, CH), jnp.int32),
            pltpu.VMEM((RB, CH), jnp.int32),
            pltpu.VMEM((CH, 128), jnp.float32),
            pltpu.VMEM((CH, 128), jnp.float32),
            pltpu.SemaphoreType.DMA,
            pltpu.SemaphoreType.DMA,
            pltpu.SemaphoreType.DMA,
            pltpu.SemaphoreType.DMA,
        ],
    )
    def k(z_hbm, zeros_hbm, src_hbm, dst_hbm, g0, g1, acc, srcv, dstv,
          rows0, rows1, gs0, gs1, ss0, ss1):
        c = lax.axis_index("c")
        s = lax.axis_index("s")
        w = c * NS + s

        my_rows = pl.ds(s * ROWS_PER_TILE, ROWS_PER_TILE)
        @pl.when(c == 0)
        def _():
            pltpu.sync_copy(z_hbm.at[my_rows], acc.at[my_rows])

        @pl.when(c == 1)
        def _():
            pltpu.sync_copy(zeros_hbm.at[my_rows], acc.at[my_rows])
        plsc.subcore_barrier()

        for b in range(nb):
            pltpu.sync_copy(src_hbm.at[w, b], srcv)
            pltpu.sync_copy(dst_hbm.at[w, b], dstv)
            _pipeline_block(z_hbm, acc, srcv, dstv, (rows0, rows1),
                            (gs0, gs1), (ss0, ss1))

        plsc.subcore_barrier()
        for cc, out in enumerate((g0, g1)):
            @pl.when(c == cc)
            def _():
                pltpu.sync_copy(acc.at[my_rows], out.at[my_rows])

    return k(z, zeros, src4, dst4)


def _sc_aggregate_cs(z0, z1, src4, dst4):
    nb = E_PAD // (NS * RB * CH)

    @functools.partial(
        pl.kernel,
        out_type=[jax.ShapeDtypeStruct((N_PAD, 128), jnp.float32)
                  for _ in range(NC)],
        mesh=_mesh(),
        scratch_types=[
            pltpu.VMEM_SHARED((N_PAD, 128), jnp.float32),
            pltpu.VMEM((RB, CH), jnp.int32),
            pltpu.VMEM((RB, CH), jnp.int32),
            pltpu.VMEM((CH, 128), jnp.float32),
            pltpu.VMEM((CH, 128), jnp.float32),
            pltpu.SemaphoreType.DMA,
            pltpu.SemaphoreType.DMA,
            pltpu.SemaphoreType.DMA,
            pltpu.SemaphoreType.DMA,
        ],
    )
    def k(z0_hbm, z1_hbm, src_hbm, dst_hbm, g0, g1, acc, srcv, dstv,
          rows0, rows1, gs0, gs1, ss0, ss1):
        c = lax.axis_index("c")
        s = lax.axis_index("s")

        my_rows = pl.ds(s * ROWS_PER_TILE, ROWS_PER_TILE)
        for cc, (zref, gout) in enumerate(((z0_hbm, g0), (z1_hbm, g1))):
            @pl.when(c == cc)
            def _():
                pltpu.sync_copy(zref.at[my_rows], acc.at[my_rows])
                plsc.subcore_barrier()
                for b in range(nb):
                    pltpu.sync_copy(src_hbm.at[s, b], srcv)
                    pltpu.sync_copy(dst_hbm.at[s, b], dstv)
                    _pipeline_block(zref, acc, srcv, dstv, (rows0, rows1),
                                    (gs0, gs1), (ss0, ss1))
                plsc.subcore_barrier()
                pltpu.sync_copy(acc.at[my_rows], gout.at[my_rows])

    return k(z0, z1, src4, dst4)


def _tc_prep(x, dega, degb):
    def body(x_ref, da_ref, db_ref, dinv_ref, z_ref):
        deg = da_ref[:, 0:1] + db_ref[:, 0:1] + 1.0
        dv = lax.rsqrt(deg)
        dinv_ref[...] = dv
        z_ref[...] = x_ref[...] * dv

    return pl.pallas_call(
        body,
        grid=(GRID,),
        in_specs=[
            pl.BlockSpec((BLK, IN_DIM), lambda i: (i, 0)),
            pl.BlockSpec((BLK, 128), lambda i: (i, 0)),
            pl.BlockSpec((BLK, 128), lambda i: (i, 0)),
        ],
        out_specs=[
            pl.BlockSpec((BLK, 1), lambda i: (i, 0)),
            pl.BlockSpec((BLK, IN_DIM), lambda i: (i, 0)),
        ],
        out_shape=[
            jax.ShapeDtypeStruct((N_NODES, 1), jnp.float32),
            jax.ShapeDtypeStruct((N_PAD, IN_DIM), jnp.float32),
        ],
    )(x, dega, degb)


def _tc_matstats(ga, gb, dinv, wt, wb, b):
    def body(ga_ref, gb_ref, dinv_ref, wt_ref, wb_ref, b_ref, u_ref, st_ref):
        i = pl.program_id(0)
        dv = dinv_ref[...]
        a0 = ga_ref[...] * dv
        a1 = gb_ref[...] * dv
        u = (jnp.dot(a0, wt_ref[...], preferred_element_type=jnp.float32,
                     precision=lax.Precision.HIGHEST)
             + jnp.dot(a1, wb_ref[...], preferred_element_type=jnp.float32,
                       precision=lax.Precision.HIGHEST)
             + b_ref[...])
        u_ref[...] = u

        @pl.when(i == 0)
        def _():
            st_ref[...] = jnp.zeros((8, HID), jnp.float32)
        st_ref[0:1, :] += jnp.sum(u, axis=0, keepdims=True)
        st_ref[1:2, :] += jnp.sum(u * u, axis=0, keepdims=True)

    return pl.pallas_call(
        body,
        grid=(GRID,),
        in_specs=[
            pl.BlockSpec((BLK, 128), lambda i: (i, 0)),
            pl.BlockSpec((BLK, 128), lambda i: (i, 0)),
            pl.BlockSpec((BLK, 1), lambda i: (i, 0)),
            pl.BlockSpec((128, HID), lambda i: (0, 0)),
            pl.BlockSpec((128, HID), lambda i: (0, 0)),
            pl.BlockSpec((1, HID), lambda i: (0, 0)),
        ],
        out_specs=[
            pl.BlockSpec((BLK, HID), lambda i: (i, 0)),
            pl.BlockSpec((8, HID), lambda i: (0, 0)),
        ],
        out_shape=[
            jax.ShapeDtypeStruct((N_NODES, HID), jnp.float32),
            jax.ShapeDtypeStruct((8, HID), jnp.float32),
        ],
    )(ga, gb, dinv, wt, wb, b)


def _tc_norm(u, st, gamma, beta, dinv):
    def body(u_ref, st_ref, g_ref, be_ref, dinv_ref, za_ref, zb_ref):
        n = jnp.float32(N_NODES)
        mean = st_ref[0:1, :] / n
        var = st_ref[1:2, :] / n - mean * mean
        inv = lax.rsqrt(var + 1e-5)
        h = (u_ref[...] - mean) * inv * g_ref[...] + be_ref[...]
        h = jnp.maximum(h, 0.0)
        z = h * dinv_ref[...]
        hh = HID // 2
        za_ref[...] = z[:, :hh]
        zb_ref[...] = z[:, hh:]

    hh = HID // 2
    return pl.pallas_call(
        body,
        grid=(GRID,),
        in_specs=[
            pl.BlockSpec((BLK, HID), lambda i: (i, 0)),
            pl.BlockSpec((8, HID), lambda i: (0, 0)),
            pl.BlockSpec((1, HID), lambda i: (0, 0)),
            pl.BlockSpec((1, HID), lambda i: (0, 0)),
            pl.BlockSpec((BLK, 1), lambda i: (i, 0)),
        ],
        out_specs=[
            pl.BlockSpec((BLK, hh), lambda i: (i, 0)),
            pl.BlockSpec((BLK, hh), lambda i: (i, 0)),
        ],
        out_shape=[
            jax.ShapeDtypeStruct((N_PAD, hh), jnp.float32),
            jax.ShapeDtypeStruct((N_PAD, hh), jnp.float32),
        ],
    )(u, st, gamma, beta, dinv)


def _tc_final(u, st, gamma, beta, wc1, bc1, wc2, bc2):
    def body(u_ref, st_ref, g_ref, be_ref, wc1_ref, bc1_ref, wc2_ref, bc2_ref,
             h_ref, gm_ref, risk_ref):
        i = pl.program_id(0)
        n = jnp.float32(N_NODES)
        mean = st_ref[0:1, :] / n
        var = st_ref[1:2, :] / n - mean * mean
        inv = lax.rsqrt(var + 1e-5)
        h = (u_ref[...] - mean) * inv * g_ref[...] + be_ref[...]
        h = jnp.maximum(h, 0.0)
        h_ref[...] = h

        @pl.when(i == 0)
        def _():
            gm_ref[...] = jnp.zeros((1, HID), jnp.float32)
        gm_ref[...] += jnp.sum(h, axis=0, keepdims=True) / n

        @pl.when(i == GRID - 1)
        def _():
            gm = gm_ref[...]
            cvec = jnp.maximum(
                jnp.dot(gm, wc1_ref[...], preferred_element_type=jnp.float32,
                        precision=lax.Precision.HIGHEST)
                + bc1_ref[...], 0.0)
            logit = (jnp.dot(cvec, wc2_ref[...],
                             preferred_element_type=jnp.float32,
                             precision=lax.Precision.HIGHEST)
                     + bc2_ref[...])
            risk_ref[...] = jax.nn.sigmoid(logit)

    return pl.pallas_call(
        body,
        grid=(GRID,),
        in_specs=[
            pl.BlockSpec((BLK, HID), lambda i: (i, 0)),
            pl.BlockSpec((8, HID), lambda i: (0, 0)),
            pl.BlockSpec((1, HID), lambda i: (0, 0)),
            pl.BlockSpec((1, HID), lambda i: (0, 0)),
            pl.BlockSpec((HID, 64), lambda i: (0, 0)),
            pl.BlockSpec((1, 64), lambda i: (0, 0)),
            pl.BlockSpec((64, 1), lambda i: (0, 0)),
            pl.BlockSpec((1, 1), lambda i: (0, 0)),
        ],
        out_specs=[
            pl.BlockSpec((BLK, HID), lambda i: (i, 0)),
            pl.BlockSpec((1, HID), lambda i: (0, 0)),
            pl.BlockSpec((1, 1), lambda i: (0, 0)),
        ],
        out_shape=[
            jax.ShapeDtypeStruct((N_NODES, HID), jnp.float32),
            jax.ShapeDtypeStruct((1, HID), jnp.float32),
            jax.ShapeDtypeStruct((1, 1), jnp.float32),
        ],
    )(u, st, gamma, beta, wc1, bc1, wc2, bc2)


def kernel(x, W1, b1, g1, be1, W2, b2, g2, be2, W3, b3, g3, be3,
           Wc1, bc1, Wc2, bc2, edge_index):
    src = edge_index[0].astype(jnp.int32)
    dst = edge_index[1].astype(jnp.int32)
    npad = E_PAD - N_EDGES
    filler = (jnp.arange(npad, dtype=jnp.int32) % 240)
    src = jnp.concatenate([src, filler])
    dst = jnp.concatenate([dst, N_NODES + filler])
    src32 = src.reshape(NC * NS, -1, RB, CH)
    dst32 = dst.reshape(NC * NS, -1, RB, CH)
    src16 = src.reshape(NS, -1, RB, CH)
    dst16 = dst.reshape(NS, -1, RB, CH)
    zeros = jnp.zeros((N_PAD, 128), jnp.float32)

    r2 = lambda v: v.reshape(1, -1)

    dega, degb = _sc_degree(dst32, zeros)
    dinv, z = _tc_prep(x, dega, degb)

    gp0, gp1 = _sc_aggregate_es(z, zeros, src32, dst32)
    u, st = _tc_matstats(gp0, gp1, dinv, W1, W1, r2(b1))
    za, zb = _tc_norm(u, st, r2(g1), r2(be1), dinv)

    ga, gb = _sc_aggregate_cs(za, zb, src16, dst16)
    u, st = _tc_matstats(ga, gb, dinv, W2[:HID // 2], W2[HID // 2:], r2(b2))
    za, zb = _tc_norm(u, st, r2(g2), r2(be2), dinv)

    ga, gb = _sc_aggregate_cs(za, zb, src16, dst16)
    u, st = _tc_matstats(ga, gb, dinv, W3[:HID // 2], W3[HID // 2:], r2(b3))
    h3, gm, risk = _tc_final(u, st, r2(g3), r2(be3), Wc1, r2(bc1), Wc2,
                             r2(bc2))

    return (risk, h3, gm)


# speedup vs baseline: 20.9650x; 1.0093x over previous
import functools

import jax
import jax.numpy as jnp
from jax import lax
from jax.experimental import pallas as pl
from jax.experimental.pallas import tpu as pltpu
from jax.experimental.pallas import tpu_sc as plsc

N_NODES = 10000
N_PAD = 10240
N_EDGES = 320000
IN_DIM = 128
HID = 256

NC = 2
NS = 16
CH = 128
RB = 40
E_PAD = 327680
ROWS_PER_TILE = N_PAD // NS
BLK = 2000
GRID = N_NODES // BLK


def _mesh():
    return plsc.VectorSubcoreMesh(core_axis_name="c", subcore_axis_name="s",
                                  num_cores=NC, num_subcores=NS)


def _pipeline_block(zref, acc, srcv, dstv, rows, gsem, ssem):
    def gather(j, k):
        pltpu.async_copy(zref.at[srcv.at[j]], rows[k], gsem[k])

    def gwait(j, k):
        pltpu.make_async_copy(zref.at[srcv.at[j]], rows[k], gsem[k]).wait()

    def scat(j, k):
        pltpu.async_copy(rows[k], acc.at[dstv.at[j]], ssem[k], add=True)

    def swait(j, k):
        pltpu.make_async_copy(rows[k], acc.at[dstv.at[j]], ssem[k]).wait()

    gather(0, 0)

    def pair(jj, _):
        j0 = jj * 2

        @pl.when(jj > 0)
        def _():
            swait(j0, 1)
        gather(j0 + 1, 1)
        gwait(j0, 0)
        scat(j0, 0)

        @pl.when(jj + 1 < RB // 2)
        def _():
            swait(j0, 0)
            gather(j0 + 2, 0)
        gwait(j0 + 1, 1)
        scat(j0 + 1, 1)
        return 0
    lax.fori_loop(0, RB // 2, pair, 0)
    swait(RB - 2, 0)
    swait(RB - 1, 1)


def _sc_degree(dst4, zeros):
    nb = E_PAD // (NC * NS * RB * CH)

    @functools.partial(
        pl.kernel,
        out_type=[jax.ShapeDtypeStruct((N_PAD, 128), jnp.float32)
                  for _ in range(NC)],
        mesh=_mesh(),
        scratch_types=[
            pltpu.VMEM_SHARED((N_PAD, 128), jnp.float32),
            pltpu.VMEM((RB, CH), jnp.int32),
            pltpu.VMEM((CH, 128), jnp.float32),
            pltpu.SemaphoreType.DMA,
        ],
    )
    def k(dst_hbm, zeros_hbm, out0, out1, acc, dstv, ones, ssem):
        c = lax.axis_index("c")
        s = lax.axis_index("s")
        w = c * NS + s

        def init_ones(i, _):
            for q in range(128 // 16):
                ones[i, pl.ds(q * 16, 16)] = jnp.ones((16,), jnp.float32)
            return 0
        lax.fori_loop(0, CH, init_ones, 0)

        my_rows = pl.ds(s * ROWS_PER_TILE, ROWS_PER_TILE)
        pltpu.sync_copy(zeros_hbm.at[my_rows], acc.at[my_rows])
        plsc.subcore_barrier()

        for b in range(nb):
            pltpu.sync_copy(dst_hbm.at[w, b], dstv)

            def fire(j, _):
                pltpu.async_copy(ones, acc.at[dstv.at[j]], ssem, add=True)
                return 0
            lax.fori_loop(0, RB, fire, 0)

            def drain(j, _):
                pltpu.make_async_copy(ones, acc.at[dstv.at[0]], ssem).wait()
                return 0
            lax.fori_loop(0, RB, drain, 0)

        plsc.subcore_barrier()
        for cc, out in enumerate((out0, out1)):
            @pl.when(c == cc)
            def _():
                pltpu.sync_copy(acc.at[my_rows], out.at[my_rows])

    return k(dst4, zeros)


def _sc_aggregate_es(z, zeros, src4, dst4):
    nb = E_PAD // (NC * NS * RB * CH)

    @functools.partial(
        pl.kernel,
        out_type=[jax.ShapeDtypeStruct((N_PAD, 128), jnp.float32)
                  for _ in range(NC)],
        mesh=_mesh(),
        scratch_types=[
            pltpu.VMEM_SHARED((N_PAD, 128), jnp.float32),
            pltpu.VMEM((RB, CH), jnp.int32),
            pltpu.VMEM((RB, CH), jnp.int32),
            pltpu.VMEM((CH, 128), jnp.float32),
            pltpu.VMEM((CH, 128), jnp.float32),
            pltpu.SemaphoreType.DMA,
            pltpu.SemaphoreType.DMA,
            pltpu.SemaphoreType.DMA,
            pltpu.SemaphoreType.DMA,
        ],
    )
    def k(z_hbm, zeros_hbm, src_hbm, dst_hbm, g0, g1, acc, srcv, dstv,
          rows0, rows1, gs0, gs1, ss0, ss1):
        c = lax.axis_index("c")
        s = lax.axis_index("s")
        w = c * NS + s

        my_rows = pl.ds(s * ROWS_PER_TILE, ROWS_PER_TILE)
        @pl.when(c == 0)
        def _():
            pltpu.sync_copy(z_hbm.at[my_rows], acc.at[my_rows])

        @pl.when(c == 1)
        def _():
            pltpu.sync_copy(zeros_hbm.at[my_rows], acc.at[my_rows])
        plsc.subcore_barrier()

        for b in range(nb):
            pltpu.sync_copy(src_hbm.at[w, b], srcv)
            pltpu.sync_copy(dst_hbm.at[w, b], dstv)
            _pipeline_block(z_hbm, acc, srcv, dstv, (rows0, rows1),
                            (gs0, gs1), (ss0, ss1))

        plsc.subcore_barrier()
        for cc, out in enumerate((g0, g1)):
            @pl.when(c == cc)
            def _():
                pltpu.sync_copy(acc.at[my_rows], out.at[my_rows])

    return k(z, zeros, src4, dst4)


def _sc_aggregate_cs(z0, z1, src4, dst4):
    nb = E_PAD // (NS * RB * CH)

    @functools.partial(
        pl.kernel,
        out_type=[jax.ShapeDtypeStruct((N_PAD, 128), jnp.float32)
                  for _ in range(NC)],
        mesh=_mesh(),
        scratch_types=[
            pltpu.VMEM_SHARED((N_PAD, 128), jnp.float32),
            pltpu.VMEM((RB, CH), jnp.int32),
            pltpu.VMEM((RB, CH), jnp.int32),
            pltpu.VMEM((CH, 128), jnp.float32),
            pltpu.VMEM((CH, 128), jnp.float32),
            pltpu.SemaphoreType.DMA,
            pltpu.SemaphoreType.DMA,
            pltpu.SemaphoreType.DMA,
            pltpu.SemaphoreType.DMA,
        ],
    )
    def k(z0_hbm, z1_hbm, src_hbm, dst_hbm, g0, g1, acc, srcv, dstv,
          rows0, rows1, gs0, gs1, ss0, ss1):
        c = lax.axis_index("c")
        s = lax.axis_index("s")

        my_rows = pl.ds(s * ROWS_PER_TILE, ROWS_PER_TILE)
        for cc, (zref, gout) in enumerate(((z0_hbm, g0), (z1_hbm, g1))):
            @pl.when(c == cc)
            def _():
                pltpu.sync_copy(zref.at[my_rows], acc.at[my_rows])
                plsc.subcore_barrier()
                for b in range(nb):
                    pltpu.sync_copy(src_hbm.at[s, b], srcv)
                    pltpu.sync_copy(dst_hbm.at[s, b], dstv)
                    _pipeline_block(zref, acc, srcv, dstv, (rows0, rows1),
                                    (gs0, gs1), (ss0, ss1))
                plsc.subcore_barrier()
                pltpu.sync_copy(acc.at[my_rows], gout.at[my_rows])

    return k(z0, z1, src4, dst4)


def _tc_prep(x, dega, degb):
    def body(x_ref, da_ref, db_ref, dinv_ref, z_ref):
        deg = da_ref[:, 0:1] + db_ref[:, 0:1] + 1.0
        dv = lax.rsqrt(deg)
        dinv_ref[...] = dv
        z_ref[...] = x_ref[...] * dv

    return pl.pallas_call(
        body,
        grid=(GRID,),
        in_specs=[
            pl.BlockSpec((BLK, IN_DIM), lambda i: (i, 0)),
            pl.BlockSpec((BLK, 128), lambda i: (i, 0)),
            pl.BlockSpec((BLK, 128), lambda i: (i, 0)),
        ],
        out_specs=[
            pl.BlockSpec((BLK, 1), lambda i: (i, 0)),
            pl.BlockSpec((BLK, IN_DIM), lambda i: (i, 0)),
        ],
        out_shape=[
            jax.ShapeDtypeStruct((N_NODES, 1), jnp.float32),
            jax.ShapeDtypeStruct((N_PAD, IN_DIM), jnp.float32),
        ],
    )(x, dega, degb)


def _tc_layer(ga, gb, dinv, wt, wb, b, gamma, beta):
    def body(ga_ref, gb_ref, dinv_ref, wt_ref, wb_ref, b_ref, g_ref, be_ref,
             za_ref, zb_ref, u_scr, st_scr):
        i = pl.program_id(0)

        @pl.when(i < GRID)
        def _():
            dv = dinv_ref[...]
            a0 = ga_ref[...] * dv
            a1 = gb_ref[...] * dv
            u = (jnp.dot(a0, wt_ref[...], preferred_element_type=jnp.float32,
                         precision=lax.Precision.HIGHEST)
                 + jnp.dot(a1, wb_ref[...], preferred_element_type=jnp.float32,
                           precision=lax.Precision.HIGHEST)
                 + b_ref[...])
            u_scr[pl.ds(lax.rem(i, GRID) * BLK, BLK), :] = u

            @pl.when(i == 0)
            def _():
                st_scr[...] = jnp.zeros((8, HID), jnp.float32)
            st_scr[0:1, :] += jnp.sum(u, axis=0, keepdims=True)
            st_scr[1:2, :] += jnp.sum(u * u, axis=0, keepdims=True)

        @pl.when(i >= GRID)
        def _():
            n = jnp.float32(N_NODES)
            mean = st_scr[0:1, :] / n
            var = st_scr[1:2, :] / n - mean * mean
            inv = lax.rsqrt(var + 1e-5)
            u = u_scr[pl.ds(lax.rem(i, GRID) * BLK, BLK), :]
            h = (u - mean) * inv * g_ref[...] + be_ref[...]
            h = jnp.maximum(h, 0.0)
            z = h * dinv_ref[...]
            hh = HID // 2
            za_ref[...] = z[:, :hh]
            zb_ref[...] = z[:, hh:]

    blk_i = lambda i: (lax.rem(i, GRID), 0)
    fix0 = lambda i: (0, 0)
    hh = HID // 2
    return pl.pallas_call(
        body,
        grid=(2 * GRID,),
        in_specs=[
            pl.BlockSpec((BLK, 128), blk_i),
            pl.BlockSpec((BLK, 128), blk_i),
            pl.BlockSpec((BLK, 1), blk_i),
            pl.BlockSpec((128, HID), fix0),
            pl.BlockSpec((128, HID), fix0),
            pl.BlockSpec((1, HID), fix0),
            pl.BlockSpec((1, HID), fix0),
            pl.BlockSpec((1, HID), fix0),
        ],
        out_specs=[
            pl.BlockSpec((BLK, hh), blk_i),
            pl.BlockSpec((BLK, hh), blk_i),
        ],
        out_shape=[
            jax.ShapeDtypeStruct((N_PAD, hh), jnp.float32),
            jax.ShapeDtypeStruct((N_PAD, hh), jnp.float32),
        ],
        scratch_shapes=[
            pltpu.VMEM((N_NODES, HID), jnp.float32),
            pltpu.VMEM((8, HID), jnp.float32),
        ],
    )(ga, gb, dinv, wt, wb, b, gamma, beta)


def _tc_final(ga, gb, dinv, wt, wb, b, gamma, beta, wc1, bc1, wc2, bc2):
    def body(ga_ref, gb_ref, dinv_ref, wt_ref, wb_ref, b_ref, g_ref, be_ref,
             wc1_ref, bc1_ref, wc2_ref, bc2_ref,
             h_ref, gm_ref, risk_ref, u_scr, st_scr):
        i = pl.program_id(0)

        @pl.when(i < GRID)
        def _():
            dv = dinv_ref[...]
            a0 = ga_ref[...] * dv
            a1 = gb_ref[...] * dv
            u = (jnp.dot(a0, wt_ref[...], preferred_element_type=jnp.float32,
                         precision=lax.Precision.HIGHEST)
                 + jnp.dot(a1, wb_ref[...], preferred_element_type=jnp.float32,
                           precision=lax.Precision.HIGHEST)
                 + b_ref[...])
            u_scr[pl.ds(lax.rem(i, GRID) * BLK, BLK), :] = u

            @pl.when(i == 0)
            def _():
                st_scr[...] = jnp.zeros((8, HID), jnp.float32)
            st_scr[0:1, :] += jnp.sum(u, axis=0, keepdims=True)
            st_scr[1:2, :] += jnp.sum(u * u, axis=0, keepdims=True)

        @pl.when(i >= GRID)
        def _():
            n = jnp.float32(N_NODES)
            mean = st_scr[0:1, :] / n
            var = st_scr[1:2, :] / n - mean * mean
            inv = lax.rsqrt(var + 1e-5)
            u = u_scr[pl.ds(lax.rem(i, GRID) * BLK, BLK), :]
            h = (u - mean) * inv * g_ref[...] + be_ref[...]
            h = jnp.maximum(h, 0.0)
            h_ref[...] = h

            @pl.when(i == GRID)
            def _():
                gm_ref[...] = jnp.zeros((1, HID), jnp.float32)
            gm_ref[...] += jnp.sum(h, axis=0, keepdims=True) / n

            @pl.when(i == 2 * GRID - 1)
            def _():
                gm = gm_ref[...]
                cvec = jnp.maximum(
                    jnp.dot(gm, wc1_ref[...],
                            preferred_element_type=jnp.float32,
                            precision=lax.Precision.HIGHEST)
                    + bc1_ref[...], 0.0)
                logit = (jnp.dot(cvec, wc2_ref[...],
                                 preferred_element_type=jnp.float32,
                                 precision=lax.Precision.HIGHEST)
                         + bc2_ref[...])
                risk_ref[...] = jax.nn.sigmoid(logit)

    blk_i = lambda i: (lax.rem(i, GRID), 0)
    fix0 = lambda i: (0, 0)
    return pl.pallas_call(
        body,
        grid=(2 * GRID,),
        in_specs=[
            pl.BlockSpec((BLK, 128), blk_i),
            pl.BlockSpec((BLK, 128), blk_i),
            pl.BlockSpec((BLK, 1), blk_i),
            pl.BlockSpec((128, HID), fix0),
            pl.BlockSpec((128, HID), fix0),
            pl.BlockSpec((1, HID), fix0),
            pl.BlockSpec((1, HID), fix0),
            pl.BlockSpec((1, HID), fix0),
            pl.BlockSpec((HID, 64), fix0),
            pl.BlockSpec((1, 64), fix0),
            pl.BlockSpec((64, 1), fix0),
            pl.BlockSpec((1, 1), fix0),
        ],
        out_specs=[
            pl.BlockSpec((BLK, HID), blk_i),
            pl.BlockSpec((1, HID), fix0),
            pl.BlockSpec((1, 1), fix0),
        ],
        out_shape=[
            jax.ShapeDtypeStruct((N_NODES, HID), jnp.float32),
            jax.ShapeDtypeStruct((1, HID), jnp.float32),
            jax.ShapeDtypeStruct((1, 1), jnp.float32),
        ],
        scratch_shapes=[
            pltpu.VMEM((N_NODES, HID), jnp.float32),
            pltpu.VMEM((8, HID), jnp.float32),
        ],
    )(ga, gb, dinv, wt, wb, b, gamma, beta, wc1, bc1, wc2, bc2)


def kernel(x, W1, b1, g1, be1, W2, b2, g2, be2, W3, b3, g3, be3,
           Wc1, bc1, Wc2, bc2, edge_index):
    src = edge_index[0].astype(jnp.int32)
    dst = edge_index[1].astype(jnp.int32)
    npad = E_PAD - N_EDGES
    filler = (jnp.arange(npad, dtype=jnp.int32) % 240)
    src = jnp.concatenate([src, filler])
    dst = jnp.concatenate([dst, N_NODES + filler])
    src32 = src.reshape(NC * NS, -1, RB, CH)
    dst32 = dst.reshape(NC * NS, -1, RB, CH)
    src16 = src.reshape(NS, -1, RB, CH)
    dst16 = dst.reshape(NS, -1, RB, CH)
    zeros = jnp.zeros((N_PAD, 128), jnp.float32)

    r2 = lambda v: v.reshape(1, -1)

    dega, degb = _sc_degree(dst32, zeros)
    dinv, z = _tc_prep(x, dega, degb)

    gp0, gp1 = _sc_aggregate_es(z, zeros, src32, dst32)
    za, zb = _tc_layer(gp0, gp1, dinv, W1, W1, r2(b1), r2(g1), r2(be1))

    ga, gb = _sc_aggregate_cs(za, zb, src16, dst16)
    za, zb = _tc_layer(ga, gb, dinv, W2[:HID // 2], W2[HID // 2:], r2(b2),
                       r2(g2), r2(be2))

    ga, gb = _sc_aggregate_cs(za, zb, src16, dst16)
    h3, gm, risk = _tc_final(ga, gb, dinv, W3[:HID // 2], W3[HID // 2:],
                             r2(b3), r2(g3), r2(be3), Wc1, r2(bc1), Wc2,
                             r2(bc2))

    return (risk, h3, gm)


# speedup vs baseline: 21.1852x; 1.0105x over previous
import functools

import jax
import jax.numpy as jnp
from jax import lax
from jax.experimental import pallas as pl
from jax.experimental.pallas import tpu as pltpu
from jax.experimental.pallas import tpu_sc as plsc

N_NODES = 10000
N_PAD = 10240
N_EDGES = 320000
IN_DIM = 128
HID = 256

NC = 2
NS = 16
CH = 128
RB = 40
E_PAD = 327680
ROWS_PER_TILE = N_PAD // NS
BLK = 2000
GRID = N_NODES // BLK


def _mesh():
    return plsc.VectorSubcoreMesh(core_axis_name="c", subcore_axis_name="s",
                                  num_cores=NC, num_subcores=NS)


def _pipeline_block(zref, acc, srcv, dstv, rows, gsem, ssem):
    def gather(j, k):
        pltpu.async_copy(zref.at[srcv.at[j]], rows[k], gsem[k])

    def gwait(j, k):
        pltpu.make_async_copy(zref.at[srcv.at[j]], rows[k], gsem[k]).wait()

    def scat(j, k):
        pltpu.async_copy(rows[k], acc.at[dstv.at[j]], ssem[k], add=True)

    def swait(j, k):
        pltpu.make_async_copy(rows[k], acc.at[dstv.at[j]], ssem[k]).wait()

    gather(0, 0)

    def pair(jj, _):
        j0 = jj * 2

        @pl.when(jj > 0)
        def _():
            swait(j0, 1)
        gather(j0 + 1, 1)
        gwait(j0, 0)
        scat(j0, 0)

        @pl.when(jj + 1 < RB // 2)
        def _():
            swait(j0, 0)
            gather(j0 + 2, 0)
        gwait(j0 + 1, 1)
        scat(j0 + 1, 1)
        return 0
    lax.fori_loop(0, RB // 2, pair, 0)
    swait(RB - 2, 0)
    swait(RB - 1, 1)


def _sc_degree(dst4, zeros):
    nb = E_PAD // (NC * NS * RB * CH)

    @functools.partial(
        pl.kernel,
        out_type=[jax.ShapeDtypeStruct((N_PAD, 128), jnp.float32)
                  for _ in range(NC)],
        mesh=_mesh(),
        scratch_types=[
            pltpu.VMEM_SHARED((N_PAD, 128), jnp.float32),
            pltpu.VMEM((RB, CH), jnp.int32),
            pltpu.VMEM((RB, CH), jnp.int32),
            pltpu.VMEM((CH, 128), jnp.float32),
            pltpu.SemaphoreType.DMA,
            pltpu.SemaphoreType.DMA,
        ],
    )
    def k(dst_hbm, zeros_hbm, out0, out1, acc, dstv, dstv2, ones, ssem,
          isem):
        c = lax.axis_index("c")
        s = lax.axis_index("s")
        w = c * NS + s

        def init_ones(i, _):
            for q in range(128 // 16):
                ones[i, pl.ds(q * 16, 16)] = jnp.ones((16,), jnp.float32)
            return 0
        lax.fori_loop(0, CH, init_ones, 0)

        my_rows = pl.ds(s * ROWS_PER_TILE, ROWS_PER_TILE)
        pltpu.sync_copy(zeros_hbm.at[my_rows], acc.at[my_rows])
        plsc.subcore_barrier()

        bufs = (dstv, dstv2)
        pltpu.sync_copy(dst_hbm.at[w, 0], dstv)
        for b in range(nb):
            buf = bufs[b % 2]
            if b + 1 < nb:
                pltpu.async_copy(dst_hbm.at[w, b + 1], bufs[(b + 1) % 2],
                                 isem)

            def fire(j, _):
                pltpu.async_copy(ones, acc.at[buf.at[j]], ssem, add=True)
                return 0
            lax.fori_loop(0, RB, fire, 0)
            if b + 1 < nb:
                pltpu.make_async_copy(dst_hbm.at[w, b + 1],
                                      bufs[(b + 1) % 2], isem).wait()

        def drain(j, _):
            pltpu.make_async_copy(ones, acc.at[dstv.at[0]], ssem).wait()
            return 0
        lax.fori_loop(0, nb * RB, drain, 0)

        plsc.subcore_barrier()
        for cc, out in enumerate((out0, out1)):
            @pl.when(c == cc)
            def _():
                pltpu.sync_copy(acc.at[my_rows], out.at[my_rows])

    return k(dst4, zeros)


def _sc_aggregate_es(z, zeros, src4, dst4):
    nb = E_PAD // (NC * NS * RB * CH)

    @functools.partial(
        pl.kernel,
        out_type=[jax.ShapeDtypeStruct((N_PAD, 128), jnp.float32)
                  for _ in range(NC)],
        mesh=_mesh(),
        scratch_types=[
            pltpu.VMEM_SHARED((N_PAD, 128), jnp.float32),
            pltpu.VMEM((RB, CH), jnp.int32),
            pltpu.VMEM((RB, CH), jnp.int32),
            pltpu.VMEM((CH, 128), jnp.float32),
            pltpu.VMEM((CH, 128), jnp.float32),
            pltpu.SemaphoreType.DMA,
            pltpu.SemaphoreType.DMA,
            pltpu.SemaphoreType.DMA,
            pltpu.SemaphoreType.DMA,
        ],
    )
    def k(z_hbm, zeros_hbm, src_hbm, dst_hbm, g0, g1, acc, srcv, dstv,
          rows0, rows1, gs0, gs1, ss0, ss1):
        c = lax.axis_index("c")
        s = lax.axis_index("s")
        w = c * NS + s

        my_rows = pl.ds(s * ROWS_PER_TILE, ROWS_PER_TILE)
        @pl.when(c == 0)
        def _():
            pltpu.sync_copy(z_hbm.at[my_rows], acc.at[my_rows])

        @pl.when(c == 1)
        def _():
            pltpu.sync_copy(zeros_hbm.at[my_rows], acc.at[my_rows])
        plsc.subcore_barrier()

        for b in range(nb):
            pltpu.async_copy(src_hbm.at[w, b], srcv, gs0)
            pltpu.async_copy(dst_hbm.at[w, b], dstv, gs1)
            pltpu.make_async_copy(src_hbm.at[w, b], srcv, gs0).wait()
            pltpu.make_async_copy(dst_hbm.at[w, b], dstv, gs1).wait()
            _pipeline_block(z_hbm, acc, srcv, dstv, (rows0, rows1),
                            (gs0, gs1), (ss0, ss1))

        plsc.subcore_barrier()
        for cc, out in enumerate((g0, g1)):
            @pl.when(c == cc)
            def _():
                pltpu.sync_copy(acc.at[my_rows], out.at[my_rows])

    return k(z, zeros, src4, dst4)


def _sc_aggregate_cs(z0, z1, src4, dst4):
    nb = E_PAD // (NS * RB * CH)

    @functools.partial(
        pl.kernel,
        out_type=[jax.ShapeDtypeStruct((N_PAD, 128), jnp.float32)
                  for _ in range(NC)],
        mesh=_mesh(),
        scratch_types=[
            pltpu.VMEM_SHARED((N_PAD, 128), jnp.float32),
            pltpu.VMEM((RB, CH), jnp.int32),
            pltpu.VMEM((RB, CH), jnp.int32),
            pltpu.VMEM((CH, 128), jnp.float32),
            pltpu.VMEM((CH, 128), jnp.float32),
            pltpu.SemaphoreType.DMA,
            pltpu.SemaphoreType.DMA,
            pltpu.SemaphoreType.DMA,
            pltpu.SemaphoreType.DMA,
        ],
    )
    def k(z0_hbm, z1_hbm, src_hbm, dst_hbm, g0, g1, acc, srcv, dstv,
          rows0, rows1, gs0, gs1, ss0, ss1):
        c = lax.axis_index("c")
        s = lax.axis_index("s")

        my_rows = pl.ds(s * ROWS_PER_TILE, ROWS_PER_TILE)
        for cc, (zref, gout) in enumerate(((z0_hbm, g0), (z1_hbm, g1))):
            @pl.when(c == cc)
            def _():
                pltpu.sync_copy(zref.at[my_rows], acc.at[my_rows])
                plsc.subcore_barrier()
                for b in range(nb):
                    pltpu.async_copy(src_hbm.at[s, b], srcv, gs0)
                    pltpu.async_copy(dst_hbm.at[s, b], dstv, gs1)
                    pltpu.make_async_copy(src_hbm.at[s, b], srcv, gs0).wait()
                    pltpu.make_async_copy(dst_hbm.at[s, b], dstv, gs1).wait()
                    _pipeline_block(zref, acc, srcv, dstv, (rows0, rows1),
                                    (gs0, gs1), (ss0, ss1))
                plsc.subcore_barrier()
                pltpu.sync_copy(acc.at[my_rows], gout.at[my_rows])

    return k(z0, z1, src4, dst4)


def _tc_prep(x, dega, degb):
    def body(x_ref, da_ref, db_ref, dinv_ref, z_ref):
        deg = da_ref[:, 0:1] + db_ref[:, 0:1] + 1.0
        dv = lax.rsqrt(deg)
        dinv_ref[...] = dv
        z_ref[...] = x_ref[...] * dv

    return pl.pallas_call(
        body,
        grid=(GRID,),
        in_specs=[
            pl.BlockSpec((BLK, IN_DIM), lambda i: (i, 0)),
            pl.BlockSpec((BLK, 128), lambda i: (i, 0)),
            pl.BlockSpec((BLK, 128), lambda i: (i, 0)),
        ],
        out_specs=[
            pl.BlockSpec((BLK, 1), lambda i: (i, 0)),
            pl.BlockSpec((BLK, IN_DIM), lambda i: (i, 0)),
        ],
        out_shape=[
            jax.ShapeDtypeStruct((N_NODES, 1), jnp.float32),
            jax.ShapeDtypeStruct((N_PAD, IN_DIM), jnp.float32),
        ],
    )(x, dega, degb)


def _tc_layer(ga, gb, dinv, wt, wb, b, gamma, beta):
    def body(ga_ref, gb_ref, dinv_ref, wt_ref, wb_ref, b_ref, g_ref, be_ref,
             za_ref, zb_ref, u_scr, st_scr):
        i = pl.program_id(0)

        @pl.when(i < GRID)
        def _():
            dv = dinv_ref[...]
            a0 = ga_ref[...] * dv
            a1 = gb_ref[...] * dv
            u = (jnp.dot(a0, wt_ref[...], preferred_element_type=jnp.float32,
                         precision=lax.Precision.HIGHEST)
                 + jnp.dot(a1, wb_ref[...], preferred_element_type=jnp.float32,
                           precision=lax.Precision.HIGHEST)
                 + b_ref[...])
            u_scr[pl.ds(lax.rem(i, GRID) * BLK, BLK), :] = u

            @pl.when(i == 0)
            def _():
                st_scr[...] = jnp.zeros((8, HID), jnp.float32)
            st_scr[0:1, :] += jnp.sum(u, axis=0, keepdims=True)
            st_scr[1:2, :] += jnp.sum(u * u, axis=0, keepdims=True)

        @pl.when(i >= GRID)
        def _():
            n = jnp.float32(N_NODES)
            mean = st_scr[0:1, :] / n
            var = st_scr[1:2, :] / n - mean * mean
            inv = lax.rsqrt(var + 1e-5)
            u = u_scr[pl.ds(lax.rem(i, GRID) * BLK, BLK), :]
            h = (u - mean) * inv * g_ref[...] + be_ref[...]
            h = jnp.maximum(h, 0.0)
            z = h * dinv_ref[...]
            hh = HID // 2
            za_ref[...] = z[:, :hh]
            zb_ref[...] = z[:, hh:]

    blk_i = lambda i: (lax.rem(i, GRID), 0)
    fix0 = lambda i: (0, 0)
    hh = HID // 2
    return pl.pallas_call(
        body,
        grid=(2 * GRID,),
        in_specs=[
            pl.BlockSpec((BLK, 128), blk_i),
            pl.BlockSpec((BLK, 128), blk_i),
            pl.BlockSpec((BLK, 1), blk_i),
            pl.BlockSpec((128, HID), fix0),
            pl.BlockSpec((128, HID), fix0),
            pl.BlockSpec((1, HID), fix0),
            pl.BlockSpec((1, HID), fix0),
            pl.BlockSpec((1, HID), fix0),
        ],
        out_specs=[
            pl.BlockSpec((BLK, hh), blk_i),
            pl.BlockSpec((BLK, hh), blk_i),
        ],
        out_shape=[
            jax.ShapeDtypeStruct((N_PAD, hh), jnp.float32),
            jax.ShapeDtypeStruct((N_PAD, hh), jnp.float32),
        ],
        scratch_shapes=[
            pltpu.VMEM((N_NODES, HID), jnp.float32),
            pltpu.VMEM((8, HID), jnp.float32),
        ],
    )(ga, gb, dinv, wt, wb, b, gamma, beta)


def _tc_final(ga, gb, dinv, wt, wb, b, gamma, beta, wc1, bc1, wc2, bc2):
    def body(ga_ref, gb_ref, dinv_ref, wt_ref, wb_ref, b_ref, g_ref, be_ref,
             wc1_ref, bc1_ref, wc2_ref, bc2_ref,
             h_ref, gm_ref, risk_ref, u_scr, st_scr):
        i = pl.program_id(0)

        @pl.when(i < GRID)
        def _():
            dv = dinv_ref[...]
            a0 = ga_ref[...] * dv
            a1 = gb_ref[...] * dv
            u = (jnp.dot(a0, wt_ref[...], preferred_element_type=jnp.float32,
                         precision=lax.Precision.HIGHEST)
                 + jnp.dot(a1, wb_ref[...], preferred_element_type=jnp.float32,
                           precision=lax.Precision.HIGHEST)
                 + b_ref[...])
            u_scr[pl.ds(lax.rem(i, GRID) * BLK, BLK), :] = u

            @pl.when(i == 0)
            def _():
                st_scr[...] = jnp.zeros((8, HID), jnp.float32)
            st_scr[0:1, :] += jnp.sum(u, axis=0, keepdims=True)
            st_scr[1:2, :] += jnp.sum(u * u, axis=0, keepdims=True)

        @pl.when(i >= GRID)
        def _():
            n = jnp.float32(N_NODES)
            mean = st_scr[0:1, :] / n
            var = st_scr[1:2, :] / n - mean * mean
            inv = lax.rsqrt(var + 1e-5)
            u = u_scr[pl.ds(lax.rem(i, GRID) * BLK, BLK), :]
            h = (u - mean) * inv * g_ref[...] + be_ref[...]
            h = jnp.maximum(h, 0.0)
            h_ref[...] = h

            @pl.when(i == GRID)
            def _():
                gm_ref[...] = jnp.zeros((1, HID), jnp.float32)
            gm_ref[...] += jnp.sum(h, axis=0, keepdims=True) / n

            @pl.when(i == 2 * GRID - 1)
            def _():
                gm = gm_ref[...]
                cvec = jnp.maximum(
                    jnp.dot(gm, wc1_ref[...],
                            preferred_element_type=jnp.float32,
                            precision=lax.Precision.HIGHEST)
                    + bc1_ref[...], 0.0)
                logit = (jnp.dot(cvec, wc2_ref[...],
                                 preferred_element_type=jnp.float32,
                                 precision=lax.Precision.HIGHEST)
                         + bc2_ref[...])
                risk_ref[...] = jax.nn.sigmoid(logit)

    blk_i = lambda i: (lax.rem(i, GRID), 0)
    fix0 = lambda i: (0, 0)
    return pl.pallas_call(
        body,
        grid=(2 * GRID,),
        in_specs=[
            pl.BlockSpec((BLK, 128), blk_i),
            pl.BlockSpec((BLK, 128), blk_i),
            pl.BlockSpec((BLK, 1), blk_i),
            pl.BlockSpec((128, HID), fix0),
            pl.BlockSpec((128, HID), fix0),
            pl.BlockSpec((1, HID), fix0),
            pl.BlockSpec((1, HID), fix0),
            pl.BlockSpec((1, HID), fix0),
            pl.BlockSpec((HID, 64), fix0),
            pl.BlockSpec((1, 64), fix0),
            pl.BlockSpec((64, 1), fix0),
            pl.BlockSpec((1, 1), fix0),
        ],
        out_specs=[
            pl.BlockSpec((BLK, HID), blk_i),
            pl.BlockSpec((1, HID), fix0),
            pl.BlockSpec((1, 1), fix0),
        ],
        out_shape=[
            jax.ShapeDtypeStruct((N_NODES, HID), jnp.float32),
            jax.ShapeDtypeStruct((1, HID), jnp.float32),
            jax.ShapeDtypeStruct((1, 1), jnp.float32),
        ],
        scratch_shapes=[
            pltpu.VMEM((N_NODES, HID), jnp.float32),
            pltpu.VMEM((8, HID), jnp.float32),
        ],
    )(ga, gb, dinv, wt, wb, b, gamma, beta, wc1, bc1, wc2, bc2)


def kernel(x, W1, b1, g1, be1, W2, b2, g2, be2, W3, b3, g3, be3,
           Wc1, bc1, Wc2, bc2, edge_index):
    src = edge_index[0].astype(jnp.int32)
    dst = edge_index[1].astype(jnp.int32)
    npad = E_PAD - N_EDGES
    filler = (jnp.arange(npad, dtype=jnp.int32) % 240)
    src = jnp.concatenate([src, filler])
    dst = jnp.concatenate([dst, N_NODES + filler])
    src32 = src.reshape(NC * NS, -1, RB, CH)
    dst32 = dst.reshape(NC * NS, -1, RB, CH)
    src16 = src.reshape(NS, -1, RB, CH)
    dst16 = dst.reshape(NS, -1, RB, CH)
    zeros = jnp.zeros((N_PAD, 128), jnp.float32)

    r2 = lambda v: v.reshape(1, -1)

    dega, degb = _sc_degree(dst32, zeros)
    dinv, z = _tc_prep(x, dega, degb)

    gp0, gp1 = _sc_aggregate_es(z, zeros, src32, dst32)
    za, zb = _tc_layer(gp0, gp1, dinv, W1, W1, r2(b1), r2(g1), r2(be1))

    ga, gb = _sc_aggregate_cs(za, zb, src16, dst16)
    za, zb = _tc_layer(ga, gb, dinv, W2[:HID // 2], W2[HID // 2:], r2(b2),
                       r2(g2), r2(be2))

    ga, gb = _sc_aggregate_cs(za, zb, src16, dst16)
    h3, gm, risk = _tc_final(ga, gb, dinv, W3[:HID // 2], W3[HID // 2:],
                             r2(b3), r2(g3), r2(be3), Wc1, r2(bc1), Wc2,
                             r2(bc2))

    return (risk, h3, gm)
